# Initial kernel scaffold; baseline (speedup 1.0000x reference)
import functools

import jax
import jax.numpy as jnp
from jax import lax
from jax.experimental import pallas as pl
from jax.experimental.pallas import tpu as pltpu
from jax.experimental.pallas import tpu_sc as plsc

B = 16384 * 200
D = 32
NW = 32
IDX_W = 128
ROWS_TOTAL = B // IDX_W
ROWS_PER_W = ROWS_TOTAL // NW
CH_ROWS = 10
C = CH_ROWS * IDX_W
NCHUNKS = ROWS_PER_W // CH_ROWS

_mesh = plsc.VectorSubcoreMesh(core_axis_name="c", subcore_axis_name="s")


@functools.partial(
    pl.kernel,
    mesh=_mesh,
    out_type=jax.ShapeDtypeStruct((B, D), jnp.float32),
    scratch_types=[
        pltpu.VMEM((CH_ROWS, IDX_W), jnp.int32),
        pltpu.VMEM((C, D), jnp.float32),
        pltpu.SemaphoreType.DMA,
    ],
)
def _sc_gather(idx_hbm, table_hbm, out_hbm, idx_v, rows_v, sem):
    wid = lax.axis_index("s") * 2 + lax.axis_index("c")
    row0 = wid * ROWS_PER_W

    def step(g, carry):
        r = row0 + g * CH_ROWS
        pltpu.sync_copy(idx_hbm.at[pl.ds(r, CH_ROWS)], idx_v)
        copies = [
            pltpu.async_copy(
                table_hbm.at[idx_v.at[j]],
                rows_v.at[pl.ds(j * IDX_W, IDX_W)],
                sem,
            )
            for j in range(CH_ROWS)
        ]
        for cp in copies:
            cp.wait()
        pltpu.sync_copy(rows_v, out_hbm.at[pl.ds(r * IDX_W, C)])
        return carry

    lax.fori_loop(0, NCHUNKS, step, 0)


def kernel(input, weight):
    idx2d = input.reshape(ROWS_TOTAL, IDX_W)
    out = _sc_gather(idx2d, weight)
    return out.reshape(input.shape[0], input.shape[1], D)


# speedup vs baseline: 4.9491x; 4.9491x over previous
import functools

import jax
import jax.numpy as jnp
from jax import lax
from jax.experimental import pallas as pl
from jax.experimental.pallas import tpu as pltpu
from jax.experimental.pallas import tpu_sc as plsc

B = 16384 * 200
D = 32
NW = 32
IDX_W = 128
ROWS_TOTAL = B // IDX_W
ROWS_PER_W = ROWS_TOTAL // NW
CH_ROWS = 16
C = CH_ROWS * IDX_W
NCHUNKS = ROWS_PER_W // CH_ROWS

_mesh = plsc.VectorSubcoreMesh(core_axis_name="c", subcore_axis_name="s")


@functools.partial(
    pl.kernel,
    mesh=_mesh,
    out_type=jax.ShapeDtypeStruct((B, D), jnp.float32),
    compiler_params=pltpu.CompilerParams(use_tc_tiling_on_sc=False),
    scratch_types=[
        pltpu.VMEM((CH_ROWS, IDX_W), jnp.int32),
        pltpu.VMEM((C, D), jnp.float32),
        pltpu.SemaphoreType.DMA,
    ],
)
def _sc_gather(idx_hbm, table_hbm, out_hbm, idx_v, rows_v, sem):
    wid = lax.axis_index("s") * 2 + lax.axis_index("c")
    row0 = wid * ROWS_PER_W

    def step(g, carry):
        r = row0 + g * CH_ROWS
        pltpu.sync_copy(idx_hbm.at[pl.ds(r, CH_ROWS)], idx_v)
        copies = [
            pltpu.async_copy(
                table_hbm.at[idx_v.at[j]],
                rows_v.at[pl.ds(j * IDX_W, IDX_W)],
                sem,
            )
            for j in range(CH_ROWS)
        ]
        for cp in copies:
            cp.wait()
        pltpu.sync_copy(rows_v, out_hbm.at[pl.ds(r * IDX_W, C)])
        return carry

    lax.fori_loop(0, NCHUNKS, step, 0)


def kernel(input, weight):
    idx2d = input.reshape(ROWS_TOTAL, IDX_W)
    out = _sc_gather(idx2d, weight)
    return out.reshape(input.shape[0], input.shape[1], D)


# speedup vs baseline: 5.0495x; 1.0203x over previous
import functools

import jax
import jax.numpy as jnp
from jax import lax
from jax.experimental import pallas as pl
from jax.experimental.pallas import tpu as pltpu
from jax.experimental.pallas import tpu_sc as plsc

B = 16384 * 200
D = 32
NW = 32
IDX_W = 128
ROWS_TOTAL = B // IDX_W
ROWS_PER_W = ROWS_TOTAL // NW
CH_ROWS = 8
C = CH_ROWS * IDX_W
NCHUNKS = ROWS_PER_W // CH_ROWS

_mesh = plsc.VectorSubcoreMesh(core_axis_name="c", subcore_axis_name="s")


@functools.partial(
    pl.kernel,
    mesh=_mesh,
    out_type=jax.ShapeDtypeStruct((B, D), jnp.float32),
    compiler_params=pltpu.CompilerParams(use_tc_tiling_on_sc=False),
    scratch_types=[
        pltpu.VMEM((CH_ROWS, IDX_W), jnp.int32),
        pltpu.VMEM((CH_ROWS, IDX_W), jnp.int32),
        pltpu.VMEM((C, D), jnp.float32),
        pltpu.VMEM((C, D), jnp.float32),
        pltpu.SemaphoreType.DMA,
        pltpu.SemaphoreType.DMA,
        pltpu.SemaphoreType.DMA,
        pltpu.SemaphoreType.DMA,
    ],
)
def _sc_gather(idx_hbm, table_hbm, out_hbm, idx0, idx1, rows0, rows1,
               gs0, gs1, ws0, ws1):
    idx_v = (idx0, idx1)
    rows_v = (rows0, rows1)
    gsem = (gs0, gs1)
    wsem = (ws0, ws1)

    wid = lax.axis_index("s") * 2 + lax.axis_index("c")
    row0 = wid * ROWS_PER_W

    def stage_idx(g, b):
        r = row0 + g * CH_ROWS
        pltpu.sync_copy(idx_hbm.at[pl.ds(r, CH_ROWS)], idx_v[b])

    def fire_gathers(b):
        for j in range(CH_ROWS):
            pltpu.async_copy(
                table_hbm.at[idx_v[b].at[j]],
                rows_v[b].at[pl.ds(j * IDX_W, IDX_W)],
                gsem[b],
            )

    def wait_gathers(b):
        pltpu.make_async_copy(table_hbm.at[pl.ds(0, C)], rows_v[b],
                              gsem[b]).wait()

    def fire_write(g, b):
        r = row0 + g * CH_ROWS
        pltpu.async_copy(rows_v[b], out_hbm.at[pl.ds(r * IDX_W, C)], wsem[b])

    def drain_write(b):
        pltpu.make_async_copy(rows_v[b], out_hbm.at[pl.ds(0, C)],
                              wsem[b]).wait()

    for b in range(2):
        stage_idx(b, b)
        fire_gathers(b)

    def step(t, carry):
        for b in range(2):
            g = 2 * t + b
            wait_gathers(b)
            fire_write(g, b)
            stage_idx(g + 2, b)
            drain_write(b)
            fire_gathers(b)
        return carry

    lax.fori_loop(0, (NCHUNKS - 2) // 2, step, 0)

    for b in range(2):
        g = NCHUNKS - 2 + b
        wait_gathers(b)
        fire_write(g, b)
        drain_write(b)


def kernel(input, weight):
    idx2d = input.reshape(ROWS_TOTAL, IDX_W)
    out = _sc_gather(idx2d, weight)
    return out.reshape(input.shape[0], input.shape[1], D)
